# baseline (speedup 1.0000x reference)
import jax
import jax.numpy as jnp
from jax.experimental import pallas as pl
from jax.experimental.pallas import tpu as pltpu
from jax.experimental.pallas import tpu_sc as plsc

_EPS = 1e-12

_VOCAB = 100000
_D = 128
_BATCH = 4096
_SEQ = 200
_N = _BATCH * _SEQ

_GW = 128
_NCHUNK = 4
_NC = _N // _NCHUNK
_BC = _BATCH // _NCHUNK


def _sc_gather(W, ids_flat):
  d = W.shape[1]
  mesh = plsc.VectorSubcoreMesh(core_axis_name="c", subcore_axis_name="s")

  @pl.kernel(
      out_type=jax.ShapeDtypeStruct((_NC, d), W.dtype),
      mesh=mesh,
      compiler_params=pltpu.CompilerParams(use_tc_tiling_on_sc=False),
  )
  def gather_kernel(w_hbm, i_hbm, o_hbm):
    def body(i_vmem, o_vmem):
      pltpu.sync_copy(w_hbm.at[i_vmem.at[0]], o_vmem)

    pltpu.emit_pipeline(
        body,
        grid=(_NC // _GW,),
        in_specs=[pl.BlockSpec((1, _GW), index_map=lambda i: (0, i))],
        out_specs=[pl.BlockSpec((_GW, d), index_map=lambda i: (i, 0))],
        core_axis_name=("c", "s"),
        dimension_semantics=(pltpu.PARALLEL,),
    )(i_hbm, o_hbm)

  return gather_kernel(W, ids_flat)


_BB = 16


def _ln_body(emb_ref, p_ref, g_ref, b_ref, o_ref):
  x = emb_ref[...].astype(jnp.float32) + p_ref[...][None, :, :]
  ones = jnp.ones((_D, _D), jnp.bfloat16)
  xb = x.astype(jnp.bfloat16).reshape(-1, _D)
  s = jnp.dot(xb, ones, preferred_element_type=jnp.float32).reshape(x.shape)
  m = s * (1.0 / _D)
  d = x - m
  db = d.astype(jnp.bfloat16)
  sq = (db * db).reshape(-1, _D)
  v = jnp.dot(sq, ones, preferred_element_type=jnp.float32).reshape(x.shape)
  scale = jax.lax.rsqrt(v * (1.0 / _D) + _EPS)
  o_ref[...] = d * scale * g_ref[0][None, None, :] + b_ref[0][None, None, :]


def _ln_body_acc(emb_ref, p_ref, g_ref, b_ref, _prev_ref, o_ref):
  _ln_body(emb_ref, p_ref, g_ref, b_ref, o_ref)


def _tc_ln_chunk(emb, W, gamma, beta, chunk, prev=None):
  base = chunk * (_BC // _BB)
  common = dict(
      grid=(_BC // _BB,),
      out_specs=pl.BlockSpec((_BB, _SEQ, _D), lambda i: (i + base, 0, 0)),
      out_shape=jax.ShapeDtypeStruct((_BATCH, _SEQ, _D), jnp.float32),
  )
  in_specs = [
      pl.BlockSpec((_BB, _SEQ, _D), lambda i: (i, 0, 0)),
      pl.BlockSpec((_SEQ, _D), lambda i: (0, 0)),
      pl.BlockSpec((1, _D), lambda i: (0, 0)),
      pl.BlockSpec((1, _D), lambda i: (0, 0)),
  ]
  if prev is None:
    return pl.pallas_call(_ln_body, in_specs=in_specs, **common)(
        emb, W, gamma, beta)
  in_specs.append(pl.BlockSpec(memory_space=pl.ANY))
  return pl.pallas_call(
      _ln_body_acc,
      in_specs=in_specs,
      input_output_aliases={4: 0},
      **common,
  )(emb, W, gamma, beta, prev)


def kernel(input_ids, W, gamma, beta):
  ids_flat = input_ids.reshape(1, _N).astype(jnp.int32)
  g2 = gamma.reshape(1, _D)
  b2 = beta.reshape(1, _D)
  Wb = W.astype(jnp.bfloat16)
  Wv = jax.lax.bitcast_convert_type(
      Wb.reshape(_VOCAB, _D // 2, 2), jnp.int32)
  out = None
  for j in range(_NCHUNK):
    idc = jax.lax.slice(ids_flat, (0, j * _NC), (1, (j + 1) * _NC))
    g32 = _sc_gather(Wv, idc)
    gb = jax.lax.bitcast_convert_type(g32, jnp.bfloat16)
    out = _tc_ln_chunk(gb.reshape(_BC, _SEQ, _D), W, g2, b2, j, out)
  return out


# speedup vs baseline: 4.1299x; 4.1299x over previous
import jax
import jax.numpy as jnp
from jax.experimental import pallas as pl
from jax.experimental.pallas import tpu as pltpu
from jax.experimental.pallas import tpu_sc as plsc

_EPS = 1e-12

_VOCAB = 100000
_D = 128
_BATCH = 4096
_SEQ = 200
_N = _BATCH * _SEQ

_GW = 128
_NCHUNK = 4
_NC = _N // _NCHUNK
_BC = _BATCH // _NCHUNK


def _sc_gather(W, ids_flat):
  d = W.shape[1]
  mesh = plsc.VectorSubcoreMesh(core_axis_name="c", subcore_axis_name="s")

  @pl.kernel(
      out_type=jax.ShapeDtypeStruct((_NC, d), W.dtype),
      mesh=mesh,
      compiler_params=pltpu.CompilerParams(use_tc_tiling_on_sc=False),
  )
  def gather_kernel(w_hbm, i_hbm, o_hbm):
    def body(i_vmem, o_vmem):
      pltpu.sync_copy(w_hbm.at[i_vmem.at[0]], o_vmem)

    pltpu.emit_pipeline(
        body,
        grid=(_NC // _GW,),
        in_specs=[pl.BlockSpec((1, _GW), index_map=lambda i: (0, i))],
        out_specs=[pl.BlockSpec((_GW, d), index_map=lambda i: (i, 0))],
        core_axis_name=("c", "s"),
        dimension_semantics=(pltpu.PARALLEL,),
    )(i_hbm, o_hbm)

  return gather_kernel(W, ids_flat)


_BB = 16


def _ln_body(emb_ref, p_ref, g_ref, b_ref, o_ref):
  w = emb_ref[...]
  lo = jax.lax.bitcast_convert_type(w << 16, jnp.float32)
  hi = jax.lax.bitcast_convert_type(w & jnp.int32(-65536), jnp.float32)
  xw = jnp.concatenate([lo, hi], axis=-1)
  x = xw + p_ref[...][None, :, :]
  ones = jnp.ones((_D, _D), jnp.bfloat16)
  xb = x.astype(jnp.bfloat16).reshape(-1, _D)
  s = jnp.dot(xb, ones, preferred_element_type=jnp.float32).reshape(x.shape)
  m = s * (1.0 / _D)
  d = x - m
  db = d.astype(jnp.bfloat16)
  sq = (db * db).reshape(-1, _D)
  v = jnp.dot(sq, ones, preferred_element_type=jnp.float32).reshape(x.shape)
  scale = jax.lax.rsqrt(v * (1.0 / _D) + _EPS)
  o_ref[...] = d * scale * g_ref[0][None, None, :] + b_ref[0][None, None, :]


def _ln_body_acc(emb_ref, p_ref, g_ref, b_ref, _prev_ref, o_ref):
  _ln_body(emb_ref, p_ref, g_ref, b_ref, o_ref)


def _tc_ln_chunk(emb, W, gamma, beta, chunk, prev=None):
  base = chunk * (_BC // _BB)
  common = dict(
      grid=(_BC // _BB,),
      out_specs=pl.BlockSpec((_BB, _SEQ, _D), lambda i: (i + base, 0, 0)),
      out_shape=jax.ShapeDtypeStruct((_BATCH, _SEQ, _D), jnp.float32),
  )
  in_specs = [
      pl.BlockSpec((_BB, _SEQ, _D // 2), lambda i: (i, 0, 0)),
      pl.BlockSpec((_SEQ, _D), lambda i: (0, 0)),
      pl.BlockSpec((1, _D), lambda i: (0, 0)),
      pl.BlockSpec((1, _D), lambda i: (0, 0)),
  ]
  if prev is None:
    return pl.pallas_call(_ln_body, in_specs=in_specs, **common)(
        emb, W, gamma, beta)
  in_specs.append(pl.BlockSpec(memory_space=pl.ANY))
  return pl.pallas_call(
      _ln_body_acc,
      in_specs=in_specs,
      input_output_aliases={4: 0},
      **common,
  )(emb, W, gamma, beta, prev)


def kernel(input_ids, W, gamma, beta):
  ids_flat = input_ids.reshape(1, _N).astype(jnp.int32)
  g2 = gamma.reshape(1, _D)
  b2 = beta.reshape(1, _D)
  lo_b = jax.lax.bitcast_convert_type(
      W[:, :64].astype(jnp.bfloat16).astype(jnp.float32), jnp.int32)
  hi_b = jax.lax.bitcast_convert_type(
      W[:, 64:].astype(jnp.bfloat16).astype(jnp.float32), jnp.int32)
  Wv = jax.lax.shift_right_logical(lo_b, 16) | hi_b
  out = None
  for j in range(_NCHUNK):
    idc = jax.lax.slice(ids_flat, (0, j * _NC), (1, (j + 1) * _NC))
    g32 = _sc_gather(Wv, idc)
    out = _tc_ln_chunk(g32.reshape(_BC, _SEQ, _D // 2), W, g2, b2, j, out)
  return out


# speedup vs baseline: 6.2996x; 1.5254x over previous
import jax
import jax.numpy as jnp
from jax.experimental import pallas as pl
from jax.experimental.pallas import tpu as pltpu
from jax.experimental.pallas import tpu_sc as plsc

_EPS = 1e-12

_VOCAB = 100000
_D = 128
_BATCH = 4096
_SEQ = 200
_N = _BATCH * _SEQ

_GW = 128
_NCHUNK = 8
_NC = _N // _NCHUNK
_BC = _BATCH // _NCHUNK


def _sc_gather(W, ids_flat):
  mesh = plsc.VectorSubcoreMesh(core_axis_name="c", subcore_axis_name="s")

  @pl.kernel(
      out_type=jax.ShapeDtypeStruct((_NC, _D), jnp.float32),
      mesh=mesh,
  )
  def gather_kernel(w_hbm, i_hbm, o_hbm):
    def body(i_vmem, o_vmem):
      pltpu.sync_copy(w_hbm.at[i_vmem.at[0]], o_vmem)

    pltpu.emit_pipeline(
        body,
        grid=(_NC // _GW,),
        in_specs=[pl.BlockSpec((1, _GW), index_map=lambda i: (0, i))],
        out_specs=[pl.BlockSpec((_GW, _D), index_map=lambda i: (i, 0))],
        core_axis_name=("c", "s"),
        dimension_semantics=(pltpu.PARALLEL,),
    )(i_hbm, o_hbm)

  return gather_kernel(W, ids_flat)


_BB = 16


def _ln_body(emb_ref, p_ref, g_ref, b_ref, o_ref):
  x = emb_ref[...] + p_ref[...][None, :, :]
  ones = jnp.ones((_D, _D), jnp.bfloat16)
  xb = x.astype(jnp.bfloat16).reshape(-1, _D)
  s = jnp.dot(xb, ones, preferred_element_type=jnp.float32).reshape(x.shape)
  m = s * (1.0 / _D)
  d = x - m
  db = d.astype(jnp.bfloat16)
  sq = (db * db).reshape(-1, _D)
  v = jnp.dot(sq, ones, preferred_element_type=jnp.float32).reshape(x.shape)
  scale = jax.lax.rsqrt(v * (1.0 / _D) + _EPS)
  o_ref[...] = d * scale * g_ref[0][None, None, :] + b_ref[0][None, None, :]


def _ln_body_acc(emb_ref, p_ref, g_ref, b_ref, _prev_ref, o_ref):
  _ln_body(emb_ref, p_ref, g_ref, b_ref, o_ref)


def _tc_ln_chunk(emb, W, gamma, beta, chunk, prev=None):
  base = chunk * (_BC // _BB)
  common = dict(
      grid=(_BC // _BB,),
      out_specs=pl.BlockSpec((_BB, _SEQ, _D), lambda i: (i + base, 0, 0)),
      out_shape=jax.ShapeDtypeStruct((_BATCH, _SEQ, _D), jnp.float32),
  )
  in_specs = [
      pl.BlockSpec((_BB, _SEQ, _D), lambda i: (i, 0, 0)),
      pl.BlockSpec((_SEQ, _D), lambda i: (0, 0)),
      pl.BlockSpec((1, _D), lambda i: (0, 0)),
      pl.BlockSpec((1, _D), lambda i: (0, 0)),
  ]
  if prev is None:
    return pl.pallas_call(_ln_body, in_specs=in_specs, **common)(
        emb, W, gamma, beta)
  in_specs.append(pl.BlockSpec(memory_space=pl.ANY))
  return pl.pallas_call(
      _ln_body_acc,
      in_specs=in_specs,
      input_output_aliases={4: 0},
      **common,
  )(emb, W, gamma, beta, prev)


def kernel(input_ids, W, gamma, beta):
  ids_flat = input_ids.reshape(1, _N).astype(jnp.int32)
  g2 = gamma.reshape(1, _D)
  b2 = beta.reshape(1, _D)
  out = None
  for j in range(_NCHUNK):
    idc = jax.lax.slice(ids_flat, (0, j * _NC), (1, (j + 1) * _NC))
    gathered = _sc_gather(W, idc)
    out = _tc_ln_chunk(gathered.reshape(_BC, _SEQ, _D), W, g2, b2, j, out)
  return out
